# Initial kernel scaffold; baseline (speedup 1.0000x reference)
import jax
import jax.numpy as jnp
from jax.experimental import pallas as pl


def kernel(x, connection_indices, W0, b0, W1, b1, W2, b2):
    raise NotImplementedError("write your pallas kernel here")


# speedup vs baseline: 3.2843x; 3.2843x over previous
import functools

import jax
import jax.numpy as jnp
from jax import lax
from jax.experimental import pallas as pl
from jax.experimental.pallas import tpu as pltpu
from jax.experimental.pallas import tpu_sc as plsc

N_IN = 100000
C = 128
K = 16
H = 32
N_OUT = 50000

NC = 2
NS = 16
NW = NC * NS

N_PER_W = 1600
N_PAD = NW * N_PER_W
CHUNK_OUT = 64
ROWS_PER_CHUNK = CHUNK_OUT * K
N_CHUNKS = N_PER_W // CHUNK_OUT
GATHER_SPLIT = 128


def _xw_body(x_ref, w_ref, o_ref):
    o_ref[...] = jnp.dot(x_ref[...], w_ref[...],
                         preferred_element_type=jnp.float32)


def _gelu_exact(v):
    return 0.5 * v * (1.0 + lax.erf(v * (2.0 ** -0.5)))


def _mlp_body(s_ref, b0_ref, w1_ref, b1_ref, w2_ref, b2_ref, o_ref):
    h0 = _gelu_exact(s_ref[...] + b0_ref[...])
    h1 = jnp.dot(h0, w1_ref[...], preferred_element_type=jnp.float32) + b1_ref[...]
    h1 = _gelu_exact(h1)
    o_ref[...] = jnp.dot(h1, w2_ref[...],
                         preferred_element_type=jnp.float32) + b2_ref[...]


def _gather_sum_body(xw_hbm, idx_hbm, out_hbm, idx_v, rows_v, acc_v, sem):
    wid = lax.axis_index("s") * NC + lax.axis_index("c")
    base_out = wid * N_PER_W

    def chunk_body(ci, carry):
        out0 = base_out + ci * CHUNK_OUT
        pltpu.sync_copy(idx_hbm.at[pl.ds(out0 * K, ROWS_PER_CHUNK)], idx_v)
        copies = []
        for g in range(ROWS_PER_CHUNK // GATHER_SPLIT):
            copies.append(pltpu.async_copy(
                xw_hbm.at[idx_v.at[pl.ds(g * GATHER_SPLIT, GATHER_SPLIT)]],
                rows_v.at[pl.ds(g * GATHER_SPLIT, GATHER_SPLIT)],
                sem))
        for cp in copies:
            cp.wait()

        def out_body(j, carry2):
            r0 = j * K
            accs = []
            for h in range(H // 16):
                acc = rows_v[r0, pl.ds(h * 16, 16)]
                for r in range(1, K):
                    acc = acc + rows_v[r0 + r, pl.ds(h * 16, 16)]
                accs.append(acc)
            for h in range(H // 16):
                acc_v[j, pl.ds(h * 16, 16)] = accs[h]
            return carry2

        lax.fori_loop(0, CHUNK_OUT, out_body, 0, unroll=2)
        pltpu.sync_copy(acc_v, out_hbm.at[pl.ds(out0, CHUNK_OUT)])
        return carry

    lax.fori_loop(0, N_CHUNKS, chunk_body, 0)


@functools.cache
def _gather_sum():
    return functools.partial(
        pl.kernel,
        out_type=jax.ShapeDtypeStruct((N_PAD, H), jnp.float32),
        mesh=plsc.VectorSubcoreMesh(core_axis_name="c", subcore_axis_name="s",
                                    num_cores=NC, num_subcores=NS),
        scratch_types=[
            pltpu.VMEM((ROWS_PER_CHUNK,), jnp.int32),
            pltpu.VMEM((ROWS_PER_CHUNK, H), jnp.float32),
            pltpu.VMEM((CHUNK_OUT, H), jnp.float32),
            pltpu.SemaphoreType.DMA,
        ],
        compiler_params=pltpu.CompilerParams(use_tc_tiling_on_sc=False),
    )(_gather_sum_body)


def kernel(x, connection_indices, W0, b0, W1, b1, W2, b2):
    B = x.shape[0]
    x2d = x.reshape(N_IN, C)

    w0m = W0.reshape(K, C, H).transpose(1, 0, 2).reshape(C, K * H)

    blk = 2000
    xw = pl.pallas_call(
        _xw_body,
        grid=(N_IN // blk,),
        in_specs=[
            pl.BlockSpec((blk, C), lambda i: (i, 0)),
            pl.BlockSpec((C, K * H), lambda i: (0, 0)),
        ],
        out_specs=pl.BlockSpec((blk, K * H), lambda i: (i, 0)),
        out_shape=jax.ShapeDtypeStruct((N_IN, K * H), jnp.float32),
    )(x2d, w0m)
    xw_flat = xw.reshape(N_IN * K, H)

    idx_flat = connection_indices * K + jnp.arange(K, dtype=jnp.int32)[None, :]
    idx_flat = jnp.pad(idx_flat, ((0, N_PAD - N_OUT), (0, 0))).reshape(N_PAD * K)

    s = _gather_sum()(xw_flat, idx_flat)[:N_OUT]

    blk2 = 2000
    out = pl.pallas_call(
        _mlp_body,
        grid=(N_OUT // blk2,),
        in_specs=[
            pl.BlockSpec((blk2, H), lambda i: (i, 0)),
            pl.BlockSpec((1, H), lambda i: (0, 0)),
            pl.BlockSpec((H, H), lambda i: (0, 0)),
            pl.BlockSpec((1, H), lambda i: (0, 0)),
            pl.BlockSpec((H, H), lambda i: (0, 0)),
            pl.BlockSpec((1, H), lambda i: (0, 0)),
        ],
        out_specs=pl.BlockSpec((blk2, H), lambda i: (i, 0)),
        out_shape=jax.ShapeDtypeStruct((N_OUT, H), jnp.float32),
    )(s, b0.reshape(1, H), W1, b1.reshape(1, H), W2, b2.reshape(1, H))

    return out.reshape(B, N_OUT, H)


# speedup vs baseline: 3.6738x; 1.1186x over previous
import functools

import jax
import jax.numpy as jnp
from jax import lax
from jax.experimental import pallas as pl
from jax.experimental.pallas import tpu as pltpu
from jax.experimental.pallas import tpu_sc as plsc

N_IN = 100000
C = 128
K = 16
H = 32
N_OUT = 50000

NC = 2
NS = 16
NW = NC * NS

N_PER_W = 1600
N_PAD = NW * N_PER_W
CHUNK_OUT = 64
ROWS_PER_CHUNK = CHUNK_OUT * K
N_CHUNKS = N_PER_W // CHUNK_OUT
GATHER_SPLIT = 128


def _xw_body(x_ref, w_ref, o_ref):
    acc = jnp.dot(x_ref[...], w_ref[...], preferred_element_type=jnp.float32)
    o_ref[...] = acc.reshape(o_ref.shape)


def _gelu_exact(v):
    return 0.5 * v * (1.0 + lax.erf(v * (2.0 ** -0.5)))


def _mlp_body(s_ref, b0_ref, w1_ref, b1_ref, w2_ref, b2_ref, o_ref):
    h0 = _gelu_exact(s_ref[...] + b0_ref[...])
    h1 = jnp.dot(h0, w1_ref[...], preferred_element_type=jnp.float32) + b1_ref[...]
    h1 = _gelu_exact(h1)
    o_ref[...] = jnp.dot(h1, w2_ref[...],
                         preferred_element_type=jnp.float32) + b2_ref[...]


def _gather_sum_body(xw_hbm, idx_hbm, out_hbm, idx_v, rows_v, acc_v, sem):
    wid = lax.axis_index("s") * NC + lax.axis_index("c")
    base_out = wid * N_PER_W

    def chunk_body(ci, carry):
        out0 = base_out + ci * CHUNK_OUT
        pltpu.sync_copy(idx_hbm.at[pl.ds(out0 * K, ROWS_PER_CHUNK)], idx_v)
        copies = []
        for g in range(ROWS_PER_CHUNK // GATHER_SPLIT):
            copies.append(pltpu.async_copy(
                xw_hbm.at[idx_v.at[pl.ds(g * GATHER_SPLIT, GATHER_SPLIT)]],
                rows_v.at[pl.ds(g * GATHER_SPLIT, GATHER_SPLIT)],
                sem))
        for cp in copies:
            cp.wait()

        def out_body(j, carry2):
            r0 = j * K
            accs = []
            for h in range(H // 16):
                acc = rows_v[r0, pl.ds(h * 16, 16)]
                for r in range(1, K):
                    acc = acc + rows_v[r0 + r, pl.ds(h * 16, 16)]
                accs.append(acc)
            for h in range(H // 16):
                acc_v[j, pl.ds(h * 16, 16)] = accs[h]
            return carry2

        lax.fori_loop(0, CHUNK_OUT, out_body, 0, unroll=2)
        pltpu.sync_copy(acc_v, out_hbm.at[pl.ds(out0, CHUNK_OUT)])
        return carry

    lax.fori_loop(0, N_CHUNKS, chunk_body, 0)


@functools.cache
def _gather_sum():
    return functools.partial(
        pl.kernel,
        out_type=jax.ShapeDtypeStruct((N_PAD, H), jnp.float32),
        mesh=plsc.VectorSubcoreMesh(core_axis_name="c", subcore_axis_name="s",
                                    num_cores=NC, num_subcores=NS),
        scratch_types=[
            pltpu.VMEM((ROWS_PER_CHUNK,), jnp.int32),
            pltpu.VMEM((ROWS_PER_CHUNK, H), jnp.float32),
            pltpu.VMEM((CHUNK_OUT, H), jnp.float32),
            pltpu.SemaphoreType.DMA,
        ],
        compiler_params=pltpu.CompilerParams(use_tc_tiling_on_sc=False),
    )(_gather_sum_body)


def kernel(x, connection_indices, W0, b0, W1, b1, W2, b2):
    B = x.shape[0]
    x2d = x.reshape(N_IN, C)

    w0m = W0.reshape(K, C, H).transpose(1, 0, 2).reshape(C, K * H)

    blk = 2000
    xw = pl.pallas_call(
        _xw_body,
        grid=(N_IN // blk,),
        in_specs=[
            pl.BlockSpec((blk, C), lambda i: (i, 0)),
            pl.BlockSpec((C, K * H), lambda i: (0, 0)),
        ],
        out_specs=pl.BlockSpec((blk * K * H,), lambda i: (i,)),
        out_shape=jax.ShapeDtypeStruct((N_IN * K * H,), jnp.float32),
    )(x2d, w0m)
    xw_flat = xw.reshape(N_IN * K, H)

    idx_flat = connection_indices * K + jnp.arange(K, dtype=jnp.int32)[None, :]
    idx_flat = jnp.pad(idx_flat, ((0, N_PAD - N_OUT), (0, 0))).reshape(N_PAD * K)

    s = _gather_sum()(xw_flat, idx_flat)[:N_OUT]

    blk2 = 2000
    out = pl.pallas_call(
        _mlp_body,
        grid=(N_OUT // blk2,),
        in_specs=[
            pl.BlockSpec((blk2, H), lambda i: (i, 0)),
            pl.BlockSpec((1, H), lambda i: (0, 0)),
            pl.BlockSpec((H, H), lambda i: (0, 0)),
            pl.BlockSpec((1, H), lambda i: (0, 0)),
            pl.BlockSpec((H, H), lambda i: (0, 0)),
            pl.BlockSpec((1, H), lambda i: (0, 0)),
        ],
        out_specs=pl.BlockSpec((blk2, H), lambda i: (i, 0)),
        out_shape=jax.ShapeDtypeStruct((N_OUT, H), jnp.float32),
    )(s, b0.reshape(1, H), W1, b1.reshape(1, H), W2, b2.reshape(1, H))

    return out.reshape(B, N_OUT, H)


# speedup vs baseline: 6.9413x; 1.8894x over previous
import functools

import jax
import jax.numpy as jnp
from jax import lax
from jax.experimental import pallas as pl
from jax.experimental.pallas import tpu as pltpu
from jax.experimental.pallas import tpu_sc as plsc

N_IN = 100000
C = 128
K = 16
H = 32
N_OUT = 50000

NC = 2
NS = 16
NW = NC * NS

N_PER_W = 1600
N_PAD = NW * N_PER_W
CHUNK_OUT = 64
ROWS_PER_CHUNK = CHUNK_OUT * K
N_CHUNKS = N_PER_W // CHUNK_OUT
GATHER_SPLIT = 128


def _xw_body(x_ref, w_ref, o_ref):
    acc = jnp.dot(x_ref[...], w_ref[...], preferred_element_type=jnp.float32)
    o_ref[...] = acc.reshape(o_ref.shape)


def _gelu_exact(v):
    return 0.5 * v * (1.0 + lax.erf(v * (2.0 ** -0.5)))


def _mlp_body(s_ref, b0_ref, w1_ref, b1_ref, w2_ref, b2_ref, o_ref):
    h0 = _gelu_exact(s_ref[...] + b0_ref[...])
    h1 = jnp.dot(h0, w1_ref[...], preferred_element_type=jnp.float32) + b1_ref[...]
    h1 = _gelu_exact(h1)
    o = jnp.dot(h1, w2_ref[...], preferred_element_type=jnp.float32) + b2_ref[...]
    o_ref[...] = o.reshape(o_ref.shape)


def _gather_sum_body(xw_hbm, conn_hbm, out_hbm, conn_v, idx_v, rows_v, acc_v,
                     sem):
    wid = lax.axis_index("s") * NC + lax.axis_index("c")
    base_out = wid * N_PER_W
    kvec = lax.iota(jnp.int32, 16)

    def chunk_body(ci, carry):
        out0 = jnp.minimum(base_out + ci * CHUNK_OUT, N_OUT - CHUNK_OUT)
        pltpu.sync_copy(conn_hbm.at[pl.ds(out0, CHUNK_OUT), :], conn_v)

        def idx_body(j, carry2):
            idx_v[pl.ds(j * K, K)] = conn_v[j, :] * K + kvec
            return carry2

        lax.fori_loop(0, CHUNK_OUT, idx_body, 0, unroll=4)

        copies = []
        for g in range(ROWS_PER_CHUNK // GATHER_SPLIT):
            copies.append(pltpu.async_copy(
                xw_hbm.at[idx_v.at[pl.ds(g * GATHER_SPLIT, GATHER_SPLIT)]],
                rows_v.at[pl.ds(g * GATHER_SPLIT, GATHER_SPLIT)],
                sem))
        for cp in copies:
            cp.wait()

        def out_body(j, carry2):
            r0 = j * K
            accs = []
            for h in range(H // 16):
                acc = rows_v[r0, pl.ds(h * 16, 16)]
                for r in range(1, K):
                    acc = acc + rows_v[r0 + r, pl.ds(h * 16, 16)]
                accs.append(acc)
            for h in range(H // 16):
                acc_v[j, pl.ds(h * 16, 16)] = accs[h]
            return carry2

        lax.fori_loop(0, CHUNK_OUT, out_body, 0, unroll=2)
        pltpu.sync_copy(acc_v, out_hbm.at[pl.ds(out0, CHUNK_OUT)])
        return carry

    lax.fori_loop(0, N_CHUNKS, chunk_body, 0)


@functools.cache
def _gather_sum():
    return functools.partial(
        pl.kernel,
        out_type=jax.ShapeDtypeStruct((N_OUT, H), jnp.float32),
        mesh=plsc.VectorSubcoreMesh(core_axis_name="c", subcore_axis_name="s",
                                    num_cores=NC, num_subcores=NS),
        scratch_types=[
            pltpu.VMEM((CHUNK_OUT, K), jnp.int32),
            pltpu.VMEM((ROWS_PER_CHUNK,), jnp.int32),
            pltpu.VMEM((ROWS_PER_CHUNK, H), jnp.float32),
            pltpu.VMEM((CHUNK_OUT, H), jnp.float32),
            pltpu.SemaphoreType.DMA,
        ],
        compiler_params=pltpu.CompilerParams(use_tc_tiling_on_sc=False),
    )(_gather_sum_body)


def kernel(x, connection_indices, W0, b0, W1, b1, W2, b2):
    B = x.shape[0]
    x2d = x.reshape(N_IN, C)

    w0m = W0.reshape(K, C, H).transpose(1, 0, 2).reshape(C, K * H)

    blk = 2000
    xw = pl.pallas_call(
        _xw_body,
        grid=(N_IN // blk,),
        in_specs=[
            pl.BlockSpec((blk, C), lambda i: (i, 0)),
            pl.BlockSpec((C, K * H), lambda i: (0, 0)),
        ],
        out_specs=pl.BlockSpec((blk * K * H,), lambda i: (i,)),
        out_shape=jax.ShapeDtypeStruct((N_IN * K * H,), jnp.float32),
    )(x2d, w0m)
    xw_flat = xw.reshape(N_IN * K, H)

    s = _gather_sum()(xw_flat, connection_indices)

    blk2 = 2000
    out = pl.pallas_call(
        _mlp_body,
        grid=(N_OUT // blk2,),
        in_specs=[
            pl.BlockSpec((blk2, H), lambda i: (i, 0)),
            pl.BlockSpec((1, H), lambda i: (0, 0)),
            pl.BlockSpec((H, H), lambda i: (0, 0)),
            pl.BlockSpec((1, H), lambda i: (0, 0)),
            pl.BlockSpec((H, H), lambda i: (0, 0)),
            pl.BlockSpec((1, H), lambda i: (0, 0)),
        ],
        out_specs=pl.BlockSpec((1, blk2, H), lambda i: (0, i, 0)),
        out_shape=jax.ShapeDtypeStruct((B, N_OUT, H), jnp.float32),
    )(s, b0.reshape(1, H), W1, b1.reshape(1, H), W2, b2.reshape(1, H))

    return out
